# Initial kernel scaffold; baseline (speedup 1.0000x reference)
import jax
import jax.numpy as jnp
from jax.experimental import pallas as pl


def kernel(x, lengths, table, W1, b1, W2, b2):
    raise NotImplementedError("write your pallas kernel here")


# speedup vs baseline: 2.0531x; 2.0531x over previous
import functools

import jax
import jax.numpy as jnp
from jax import lax
from jax.experimental import pallas as pl
from jax.experimental.pallas import tpu as pltpu
from jax.experimental.pallas import tpu_sc as plsc

EMB = 32
HID = 128
NCLS = 10
B = 4096
L = 200
HALF = L // 2
NW = 32
SEQ_PER_W = B // NW


def _pool_body(x_hbm, table_hbm, out_hbm, idx_v, rows_v, sum_v, sem):
    c = lax.axis_index("c")
    s = lax.axis_index("s")
    wid = s * 2 + c
    base = wid * SEQ_PER_W

    pltpu.sync_copy(x_hbm.at[pl.ds(base, SEQ_PER_W)], idx_v)

    def seq_body(i, _):
        cp0 = pltpu.async_copy(
            table_hbm.at[idx_v.at[i, 0]], rows_v.at[pl.ds(0, HALF)], sem)
        cp1 = pltpu.async_copy(
            table_hbm.at[idx_v.at[i, 1]], rows_v.at[pl.ds(HALF, HALF)], sem)
        cp0.wait()
        cp1.wait()

        def red(t, carry):
            a0, a1 = carry
            for u in range(8):
                r = t * 8 + u
                a0 = a0 + rows_v[r, pl.ds(0, 16)]
                a1 = a1 + rows_v[r, pl.ds(16, 16)]
            return a0, a1

        z = jnp.zeros((16,), jnp.float32)
        a0, a1 = lax.fori_loop(0, L // 8, red, (z, z))
        sum_v[i, pl.ds(0, 16)] = a0
        sum_v[i, pl.ds(16, 16)] = a1
        return 0

    lax.fori_loop(0, SEQ_PER_W, seq_body, 0)
    pltpu.sync_copy(sum_v, out_hbm.at[pl.ds(base, SEQ_PER_W)])


@functools.partial(jax.jit, static_argnames=())
def _pool(x3, table):
    mesh = plsc.VectorSubcoreMesh(core_axis_name="c", subcore_axis_name="s")
    k = pl.kernel(
        _pool_body,
        out_type=jax.ShapeDtypeStruct((B, EMB), jnp.float32),
        mesh=mesh,
        scratch_types=[
            pltpu.VMEM((SEQ_PER_W, 2, HALF), jnp.int32),
            pltpu.VMEM((L, EMB), jnp.float32),
            pltpu.VMEM((SEQ_PER_W, EMB), jnp.float32),
            pltpu.SemaphoreType.DMA,
        ],
        compiler_params=pltpu.CompilerParams(use_tc_tiling_on_sc=False),
    )
    return k(x3, table)


def _mlp_body(sum_ref, len_ref, w1_ref, b1_ref, w2_ref, b2_ref, out_ref):
    lens = jnp.maximum(len_ref[...].astype(jnp.float32), 1.0)
    avg = sum_ref[...] / lens
    h = lax.dot_general(avg, w1_ref[...], (((1,), (1,)), ((), ())),
                        preferred_element_type=jnp.float32) + b1_ref[...]
    h = jnp.maximum(h, 0.0)
    out_ref[...] = lax.dot_general(h, w2_ref[...], (((1,), (1,)), ((), ())),
                                   preferred_element_type=jnp.float32) + b2_ref[...]


def _mlp(summed, lengths, W1, b1, W2, b2):
    return pl.pallas_call(
        _mlp_body,
        out_shape=jax.ShapeDtypeStruct((B, NCLS), jnp.float32),
    )(summed, lengths.reshape(B, 1), W1, b1.reshape(1, HID), W2,
      b2.reshape(1, NCLS))


def kernel(x, lengths, table, W1, b1, W2, b2):
    x3 = x.reshape(B, 2, HALF)
    summed = _pool(x3, table)
    return _mlp(summed, lengths, W1, b1, W2, b2)
